# baseline (speedup 1.0000x reference)
import functools

import jax
import jax.numpy as jnp
from jax import lax
from jax.experimental import pallas as pl
from jax.experimental.pallas import tpu as pltpu
from jax.experimental.pallas import tpu_sc as plsc

_B = 2
_NA = 16384
_NR = 2048
_CA = 128
_CS = 384

_BLK = 2048
_CHK = 256
_NAP = _NA + _BLK
_NT = 16
_APT = _NA // _NT
_NV = _APT // 16
_RPT = _NR // _NT
_QR = 32
_CG = _CS // 16


def _proj_body(x_ref, m_ref, g_ref, b_ref, w_ref, o_ref, carry_ref):
    k = pl.program_id(1)
    nblk = _NAP // _BLK

    @pl.when(k == 0)
    def _():
        carry_ref[...] = jnp.zeros_like(carry_ref)

    @pl.when(k < nblk - 1)
    def _():
        x = x_ref[0]
        mu = jnp.mean(x, axis=-1, keepdims=True)
        var = jnp.mean((x - mu) ** 2, axis=-1, keepdims=True)
        h = (x - mu) * lax.rsqrt(var + 1e-5) * g_ref[...] + b_ref[...]
        y = jnp.maximum(
            jnp.dot(h.astype(jnp.bfloat16), w_ref[...].astype(jnp.bfloat16),
                    preferred_element_type=jnp.float32),
            0.0)
        ii = lax.broadcasted_iota(jnp.int32, (_CHK, _CHK), 0)
        jj = lax.broadcasted_iota(jnp.int32, (_CHK, _CHK), 1)
        lt = (jj < ii).astype(jnp.float32)
        run = carry_ref[...]
        for ck in range(_BLK // _CHK):
            mrow = m_ref[0, ck:ck + 1, :]
            ltm = (lt * mrow).astype(jnp.bfloat16)
            ych = y[ck * _CHK:(ck + 1) * _CHK].astype(jnp.bfloat16)
            ex = jnp.dot(ltm, ych, preferred_element_type=jnp.float32)
            o_ref[0, ck * _CHK:(ck + 1) * _CHK, :] = ex + run
            run = run + (ex[_CHK - 1:_CHK, :]
                         + ych[_CHK - 1:_CHK, :].astype(jnp.float32)
                         * m_ref[0, ck:ck + 1, _CHK - 1:_CHK])
        carry_ref[...] = run

    @pl.when(k == nblk - 1)
    def _():
        o_ref[0] = jnp.broadcast_to(carry_ref[...], (_BLK, _CS))


def _prefix(x3d, mask3d, gamma, beta, w):
    nreal = _NA // _BLK - 1
    return pl.pallas_call(
        _proj_body,
        grid=(_B, _NAP // _BLK),
        in_specs=[
            pl.BlockSpec((1, _BLK, _CA),
                         lambda b, k: (b, jnp.minimum(k, nreal), 0)),
            pl.BlockSpec((1, _BLK // _CHK, _CHK),
                         lambda b, k: (b, jnp.minimum(k, nreal), 0)),
            pl.BlockSpec((1, _CA), lambda b, k: (0, 0)),
            pl.BlockSpec((1, _CA), lambda b, k: (0, 0)),
            pl.BlockSpec((_CA, _CS), lambda b, k: (0, 0)),
        ],
        out_specs=pl.BlockSpec((1, _BLK, _CS), lambda b, k: (b, k, 0)),
        out_shape=jax.ShapeDtypeStruct((_B, _NAP, _CS), jnp.float32),
        scratch_shapes=[pltpu.VMEM((1, _CS), jnp.float32)],
    )(x3d, mask3d, gamma, beta, w)


_mesh = plsc.VectorSubcoreMesh(
    core_axis_name="c", subcore_axis_name="s", num_cores=2, num_subcores=_NT
)

def _splat(vec, lane):
    return vec.at[jnp.full((16,), lane, jnp.int32)].get(
        mode="promise_in_bounds")


@functools.partial(
    pl.kernel,
    out_type=(
        jax.ShapeDtypeStruct((_B, _NR), jnp.int32),
        jax.ShapeDtypeStruct((_B, _NR), jnp.int32),
        jax.ShapeDtypeStruct((_B, _NR), jnp.float32),
    ),
    mesh=_mesh,
    compiler_params=pltpu.CompilerParams(needs_layout_passes=False),
    scratch_types=[
        pltpu.VMEM((_APT + 16,), jnp.int32),
        pltpu.VMEM((16,), jnp.int32),
        pltpu.VMEM((_NR,), jnp.int32),
        pltpu.VMEM((_NR,), jnp.int32),
        pltpu.VMEM((_NT, _RPT), jnp.int32),
        pltpu.VMEM((_NT, _RPT), jnp.int32),
        pltpu.VMEM((_RPT,), jnp.int32),
        pltpu.VMEM((_RPT,), jnp.int32),
        pltpu.VMEM((_RPT,), jnp.float32),
        pltpu.VMEM_SHARED((_NT, _NR), jnp.int32),
        pltpu.VMEM_SHARED((_NT, _NR), jnp.int32),
    ],
)
def _sc_tables(idx, gfo, glo, rco, idx_v, pn_v, f_loc, l_loc, fbuf, lbuf,
               gf, gl, recip, fsh, lsh):
    c = lax.axis_index("c")
    sid = lax.axis_index("s")
    i16 = jnp.arange(16, dtype=jnp.int32)
    base = sid * _APT
    r0 = sid * _RPT

    def _zero(i, _):
        z = jnp.zeros((16,), jnp.int32)
        f_loc[pl.ds(i * 16, 16)] = z
        l_loc[pl.ds(i * 16, 16)] = z
        return 0

    lax.fori_loop(0, _NR // 16, _zero, 0)

    pltpu.sync_copy(idx.at[c, pl.ds(base, _APT)], idx_v.at[pl.ds(0, _APT)])
    pn_v[...] = jnp.full((16,), -1, jnp.int32)
    idx_v[pl.ds(_APT, 16)] = jnp.full((16,), -2, jnp.int32)

    @pl.when(sid > 0)
    def _():
        pltpu.sync_copy(idx.at[c, pl.ds(base - 16, 16)], pn_v)

    @pl.when(sid < _NT - 1)
    def _():
        pltpu.sync_copy(idx.at[c, pl.ds(base + _APT, 16)],
                        idx_v.at[pl.ds(_APT, 16)])

    def _rdet(v, prev_vec):
        iv = idx_v[pl.ds(v * 16, 16)]
        nxt_vec = idx_v[pl.ds(v * 16 + 16, 16)]
        sh = iv.at[jnp.maximum(i16 - 1, 0)].get(mode="promise_in_bounds")
        sh = jnp.where(i16 == 0, _splat(prev_vec, 15), sh)
        nh = iv.at[jnp.minimum(i16 + 1, 15)].get(mode="promise_in_bounds")
        nh = jnp.where(i16 == 15, _splat(nxt_vec, 0), nh)
        posv = jnp.full((16,), base, jnp.int32) + v * 16 + i16
        plsc.store_scatter(f_loc, [iv], posv, mask=iv != sh)
        plsc.store_scatter(l_loc, [iv], posv + 1, mask=iv != nh)
        return iv

    lax.fori_loop(0, _NV, _rdet, pn_v[...])

    pltpu.sync_copy(f_loc, fsh.at[sid])
    pltpu.sync_copy(l_loc, lsh.at[sid])
    plsc.subcore_barrier()
    pltpu.sync_copy(fsh.at[:, pl.ds(r0, _RPT)], fbuf)
    pltpu.sync_copy(lsh.at[:, pl.ds(r0, _RPT)], lbuf)

    prow = c * _NAP

    def _combine(g, _):
        accf = jnp.zeros((16,), jnp.int32)
        accl = jnp.zeros((16,), jnp.int32)
        for t in range(_NT):
            accf = accf + fbuf[t, pl.ds(g * 16, 16)]
            accl = accl + lbuf[t, pl.ds(g * 16, 16)]
        cntf = (accl - accf).astype(jnp.float32)
        recip[pl.ds(g * 16, 16)] = 1.0 / (1.0 + cntf)
        off = jnp.full((16,), 0, jnp.int32) + prow
        gf[pl.ds(g * 16, 16)] = accf + off
        gl[pl.ds(g * 16, 16)] = accl + off
        return 0

    lax.fori_loop(0, _RPT // 16, _combine, 0)
    pltpu.sync_copy(gf, gfo.at[c, pl.ds(r0, _RPT)])
    pltpu.sync_copy(gl, glo.at[c, pl.ds(r0, _RPT)])
    pltpu.sync_copy(recip, rco.at[c, pl.ds(r0, _RPT)])


@functools.partial(
    pl.kernel,
    out_type=jax.ShapeDtypeStruct((_B, _NR, _CS), jnp.float32),
    mesh=_mesh,
    compiler_params=pltpu.CompilerParams(needs_layout_passes=False),
    scratch_types=[
        pltpu.VMEM((_RPT,), jnp.int32),
        pltpu.VMEM((_RPT,), jnp.int32),
        pltpu.VMEM((_RPT,), jnp.float32),
        pltpu.VMEM((_QR, _CS), jnp.float32),
        pltpu.VMEM((_QR, _CS), jnp.float32),
        pltpu.VMEM((_QR, _CS), jnp.float32),
        pltpu.VMEM((_QR, _CS), jnp.float32),
        pltpu.VMEM((_QR, _CS), jnp.float32),
        pltpu.VMEM((_QR, _CS), jnp.float32),
        pltpu.SemaphoreType.DMA,
        pltpu.SemaphoreType.DMA,
        pltpu.SemaphoreType.DMA,
        pltpu.SemaphoreType.DMA,
    ],
)
def _sc_finalize(p2d, s, gfo, glo, rco, out, gf, gl, recip,
                 pg1a, pg1b, pg2a, pg2b, sra, srb, semg0, semg1, semo0, semo1):
    c = lax.axis_index("c")
    sid = lax.axis_index("s")
    r0 = sid * _RPT
    pltpu.sync_copy(gfo.at[c, pl.ds(r0, _RPT)], gf)
    pltpu.sync_copy(glo.at[c, pl.ds(r0, _RPT)], gl)
    pltpu.sync_copy(rco.at[c, pl.ds(r0, _RPT)], recip)

    bufs = ((pg1a, pg2a, sra, semg0, semo0), (pg1b, pg2b, srb, semg1, semo1))
    nq = _RPT // _QR

    def _issue(q):
        b1, b2, bs, sg, _ = bufs[q % 2]
        return (
            pltpu.async_copy(p2d.at[gf.at[pl.ds(q * _QR, _QR)]], b1, sg),
            pltpu.async_copy(p2d.at[gl.at[pl.ds(q * _QR, _QR)]], b2, sg),
            pltpu.async_copy(s.at[c, pl.ds(r0 + q * _QR, _QR)], bs, sg),
        )

    pend = {0: _issue(0)}
    outh = {}
    for q in range(nq):
        b1, b2, bs, _, so = bufs[q % 2]
        if q + 1 < nq:
            if (q + 1) % 2 in outh:
                outh.pop((q + 1) % 2).wait()
            pend[q + 1] = _issue(q + 1)
        for hh in pend.pop(q):
            hh.wait()

        def _rowfin(i, _, q=q, b1=b1, b2=b2, bs=bs):
            g0 = q * _QR + i * 8
            rv = recip[pl.ds((g0 // 16) * 16, 16)]
            for u in range(8):
                r = i * 8 + u
                rs = rv.at[jnp.full((16,), (g0 + u) % 16, jnp.int32)].get(
                    mode="promise_in_bounds")
                for cg in range(_CG):
                    sl = pl.ds(cg * 16, 16)
                    bs[r, sl] = (bs[r, sl] + b2[r, sl] - b1[r, sl]) * rs
            return 0

        lax.fori_loop(0, _QR // 8, _rowfin, 0)
        outh[q % 2] = pltpu.async_copy(bs, out.at[c, pl.ds(r0 + q * _QR, _QR)],
                                       so)
    for hh in outh.values():
        hh.wait()


def kernel(atom_features, s, atom_res_idx, atom_mask, ln_gamma, ln_beta, W):
    idx = atom_res_idx.astype(jnp.int32)
    gfo, glo, rco = _sc_tables(idx)
    p = _prefix(atom_features, atom_mask.reshape(_B, _NA // _CHK, _CHK),
                ln_gamma.reshape(1, _CA), ln_beta.reshape(1, _CA), W)
    return _sc_finalize(p.reshape(_B * _NAP, _CS), s, gfo, glo, rco)


# speedup vs baseline: 1.1509x; 1.1509x over previous
import functools

import jax
import jax.numpy as jnp
from jax import lax
from jax.experimental import pallas as pl
from jax.experimental.pallas import tpu as pltpu
from jax.experimental.pallas import tpu_sc as plsc

_B = 2
_NA = 16384
_NR = 2048
_CA = 128
_CS = 384

_BLK = 2048
_CHK = 256
_NAP = _NA + _BLK
_NT = 16
_APT = _NA // _NT
_NV = _APT // 16
_RPT = _NR // _NT
_QR = 32
_CG = _CS // 16


def _proj_body(x_ref, m_ref, g_ref, b_ref, w_ref, o_ref, carry_ref):
    k = pl.program_id(1)
    nblk = _NAP // _BLK

    @pl.when(k == 0)
    def _():
        carry_ref[...] = jnp.zeros_like(carry_ref)

    @pl.when(k < nblk - 1)
    def _():
        x = x_ref[0]
        mu = jnp.mean(x, axis=-1, keepdims=True)
        var = jnp.mean((x - mu) ** 2, axis=-1, keepdims=True)
        h = (x - mu) * lax.rsqrt(var + 1e-5) * g_ref[...] + b_ref[...]
        y = jnp.maximum(
            jnp.dot(h.astype(jnp.bfloat16), w_ref[...].astype(jnp.bfloat16),
                    preferred_element_type=jnp.float32),
            0.0)
        ii = lax.broadcasted_iota(jnp.int32, (_CHK, _CHK), 0)
        jj = lax.broadcasted_iota(jnp.int32, (_CHK, _CHK), 1)
        lt = (jj < ii).astype(jnp.float32)
        run = carry_ref[...]
        for ck in range(_BLK // _CHK):
            mrow = m_ref[0, ck:ck + 1, :]
            ltm = (lt * mrow).astype(jnp.bfloat16)
            ych = y[ck * _CHK:(ck + 1) * _CHK].astype(jnp.bfloat16)
            ex = jnp.dot(ltm, ych, preferred_element_type=jnp.float32)
            o_ref[0, ck * _CHK:(ck + 1) * _CHK, :] = ex + run
            run = run + (ex[_CHK - 1:_CHK, :]
                         + ych[_CHK - 1:_CHK, :].astype(jnp.float32)
                         * m_ref[0, ck:ck + 1, _CHK - 1:_CHK])
        carry_ref[...] = run

    @pl.when(k == nblk - 1)
    def _():
        o_ref[0] = jnp.broadcast_to(carry_ref[...], (_BLK, _CS))


def _prefix(x3d, mask3d, gamma, beta, w):
    nreal = _NA // _BLK - 1
    return pl.pallas_call(
        _proj_body,
        grid=(_B, _NAP // _BLK),
        in_specs=[
            pl.BlockSpec((1, _BLK, _CA),
                         lambda b, k: (b, jnp.minimum(k, nreal), 0)),
            pl.BlockSpec((1, _BLK // _CHK, _CHK),
                         lambda b, k: (b, jnp.minimum(k, nreal), 0)),
            pl.BlockSpec((1, _CA), lambda b, k: (0, 0)),
            pl.BlockSpec((1, _CA), lambda b, k: (0, 0)),
            pl.BlockSpec((_CA, _CS), lambda b, k: (0, 0)),
        ],
        out_specs=pl.BlockSpec((1, _BLK, _CS), lambda b, k: (b, k, 0)),
        out_shape=jax.ShapeDtypeStruct((_B, _NAP, _CS), jnp.float32),
        scratch_shapes=[pltpu.VMEM((1, _CS), jnp.float32)],
    )(x3d, mask3d, gamma, beta, w)


_mesh = plsc.VectorSubcoreMesh(
    core_axis_name="c", subcore_axis_name="s", num_cores=2, num_subcores=_NT
)

def _splat(vec, lane):
    return vec.at[jnp.full((16,), lane, jnp.int32)].get(
        mode="promise_in_bounds")


@functools.partial(
    pl.kernel,
    out_type=(
        jax.ShapeDtypeStruct((_B, _NR), jnp.int32),
        jax.ShapeDtypeStruct((_B, _NR), jnp.int32),
        jax.ShapeDtypeStruct((_B, _NR), jnp.float32),
    ),
    mesh=_mesh,
    compiler_params=pltpu.CompilerParams(needs_layout_passes=False),
    scratch_types=[
        pltpu.VMEM((_APT + 16,), jnp.int32),
        pltpu.VMEM((16,), jnp.int32),
        pltpu.VMEM((_NR,), jnp.int32),
        pltpu.VMEM((_NR,), jnp.int32),
        pltpu.VMEM((_NT, _RPT), jnp.int32),
        pltpu.VMEM((_NT, _RPT), jnp.int32),
        pltpu.VMEM((_RPT,), jnp.int32),
        pltpu.VMEM((_RPT,), jnp.int32),
        pltpu.VMEM((_RPT,), jnp.float32),
        pltpu.VMEM_SHARED((_NT, _NR), jnp.int32),
        pltpu.VMEM_SHARED((_NT, _NR), jnp.int32),
    ],
)
def _sc_tables(idx, gfo, glo, rco, idx_v, pn_v, f_loc, l_loc, fbuf, lbuf,
               gf, gl, recip, fsh, lsh):
    c = lax.axis_index("c")
    sid = lax.axis_index("s")
    i16 = jnp.arange(16, dtype=jnp.int32)
    base = sid * _APT
    r0 = sid * _RPT

    def _zero(i, _):
        z = jnp.zeros((16,), jnp.int32)
        f_loc[pl.ds(i * 16, 16)] = z
        l_loc[pl.ds(i * 16, 16)] = z
        return 0

    lax.fori_loop(0, _NR // 16, _zero, 0)

    pltpu.sync_copy(idx.at[c, pl.ds(base, _APT)], idx_v.at[pl.ds(0, _APT)])
    pn_v[...] = jnp.full((16,), -1, jnp.int32)
    idx_v[pl.ds(_APT, 16)] = jnp.full((16,), -2, jnp.int32)

    @pl.when(sid > 0)
    def _():
        pltpu.sync_copy(idx.at[c, pl.ds(base - 16, 16)], pn_v)

    @pl.when(sid < _NT - 1)
    def _():
        pltpu.sync_copy(idx.at[c, pl.ds(base + _APT, 16)],
                        idx_v.at[pl.ds(_APT, 16)])

    def _rdet(v, prev_vec):
        iv = idx_v[pl.ds(v * 16, 16)]
        nxt_vec = idx_v[pl.ds(v * 16 + 16, 16)]
        sh = iv.at[jnp.maximum(i16 - 1, 0)].get(mode="promise_in_bounds")
        sh = jnp.where(i16 == 0, _splat(prev_vec, 15), sh)
        nh = iv.at[jnp.minimum(i16 + 1, 15)].get(mode="promise_in_bounds")
        nh = jnp.where(i16 == 15, _splat(nxt_vec, 0), nh)
        posv = jnp.full((16,), base, jnp.int32) + v * 16 + i16
        plsc.store_scatter(f_loc, [iv], posv, mask=iv != sh)
        plsc.store_scatter(l_loc, [iv], posv + 1, mask=iv != nh)
        return iv

    lax.fori_loop(0, _NV, _rdet, pn_v[...])

    pltpu.sync_copy(f_loc, fsh.at[sid])
    pltpu.sync_copy(l_loc, lsh.at[sid])
    plsc.subcore_barrier()
    pltpu.sync_copy(fsh.at[:, pl.ds(r0, _RPT)], fbuf)
    pltpu.sync_copy(lsh.at[:, pl.ds(r0, _RPT)], lbuf)

    prow = c * _NAP

    def _combine(g, _):
        accf = jnp.zeros((16,), jnp.int32)
        accl = jnp.zeros((16,), jnp.int32)
        for t in range(_NT):
            accf = accf + fbuf[t, pl.ds(g * 16, 16)]
            accl = accl + lbuf[t, pl.ds(g * 16, 16)]
        cntf = (accl - accf).astype(jnp.float32)
        recip[pl.ds(g * 16, 16)] = 1.0 / (1.0 + cntf)
        off = jnp.full((16,), 0, jnp.int32) + prow
        gf[pl.ds(g * 16, 16)] = accf + off
        gl[pl.ds(g * 16, 16)] = accl + off
        return 0

    lax.fori_loop(0, _RPT // 16, _combine, 0)
    pltpu.sync_copy(gf, gfo.at[c, pl.ds(r0, _RPT)])
    pltpu.sync_copy(gl, glo.at[c, pl.ds(r0, _RPT)])
    pltpu.sync_copy(recip, rco.at[c, pl.ds(r0, _RPT)])


@functools.partial(
    pl.kernel,
    out_type=jax.ShapeDtypeStruct((_B, _NR, _CS), jnp.float32),
    mesh=_mesh,
    compiler_params=pltpu.CompilerParams(needs_layout_passes=False),
    scratch_types=[
        pltpu.VMEM((_RPT,), jnp.int32),
        pltpu.VMEM((_RPT,), jnp.int32),
        pltpu.VMEM((_RPT,), jnp.float32),
        pltpu.VMEM((_QR, _CS), jnp.float32),
        pltpu.VMEM((_QR, _CS), jnp.float32),
        pltpu.VMEM((_QR, _CS), jnp.float32),
        pltpu.VMEM((_QR, _CS), jnp.float32),
        pltpu.VMEM((_QR, _CS), jnp.float32),
        pltpu.VMEM((_QR, _CS), jnp.float32),
        pltpu.SemaphoreType.DMA,
        pltpu.SemaphoreType.DMA,
        pltpu.SemaphoreType.DMA,
        pltpu.SemaphoreType.DMA,
    ],
)
def _sc_finalize(p2d, s, gfo, glo, rco, out, gf, gl, recip,
                 pg1a, pg1b, pg2a, pg2b, sra, srb, semg0, semg1, semo0, semo1):
    c = lax.axis_index("c")
    sid = lax.axis_index("s")
    r0 = sid * _RPT
    pltpu.sync_copy(gfo.at[c, pl.ds(r0, _RPT)], gf)
    pltpu.sync_copy(glo.at[c, pl.ds(r0, _RPT)], gl)
    pltpu.sync_copy(rco.at[c, pl.ds(r0, _RPT)], recip)

    bufs = ((pg1a, pg2a, sra, semg0, semo0), (pg1b, pg2b, srb, semg1, semo1))
    nq = _RPT // _QR

    def _issue(q):
        b1, b2, bs, sg, _ = bufs[q % 2]
        return (
            pltpu.async_copy(p2d.at[gf.at[pl.ds(q * _QR, _QR)]], b1, sg),
            pltpu.async_copy(p2d.at[gl.at[pl.ds(q * _QR, _QR)]], b2, sg),
            pltpu.async_copy(s.at[c, pl.ds(r0 + q * _QR, _QR)], bs, sg),
        )

    pend = {0: _issue(0)}
    outh = {}
    for q in range(nq):
        b1, b2, bs, _, so = bufs[q % 2]
        if q + 1 < nq:
            if (q + 1) % 2 in outh:
                outh.pop((q + 1) % 2).wait()
            pend[q + 1] = _issue(q + 1)
        for hh in pend.pop(q):
            hh.wait()

        def _rowfin(i, _, q=q, b1=b1, b2=b2, bs=bs):
            for u in range(2):
                r = i * 2 + u
                qq = q * _QR + r
                rv = recip[pl.ds((qq // 16) * 16, 16)]
                rs = rv.at[jnp.full((16,), qq % 16, jnp.int32)].get(
                    mode="promise_in_bounds")
                for cg in range(_CG):
                    sl = pl.ds(cg * 16, 16)
                    bs[r, sl] = (bs[r, sl] + b2[r, sl] - b1[r, sl]) * rs
            return 0

        lax.fori_loop(0, _QR // 2, _rowfin, 0)
        outh[q % 2] = pltpu.async_copy(bs, out.at[c, pl.ds(r0 + q * _QR, _QR)],
                                       so)
    for hh in outh.values():
        hh.wait()


def kernel(atom_features, s, atom_res_idx, atom_mask, ln_gamma, ln_beta, W):
    idx = atom_res_idx.astype(jnp.int32)
    gfo, glo, rco = _sc_tables(idx)
    p = _prefix(atom_features, atom_mask.reshape(_B, _NA // _CHK, _CHK),
                ln_gamma.reshape(1, _CA), ln_beta.reshape(1, _CA), W)
    return _sc_finalize(p.reshape(_B * _NAP, _CS), s, gfo, glo, rco)


# speedup vs baseline: 1.1538x; 1.0025x over previous
import functools

import jax
import jax.numpy as jnp
from jax import lax
from jax.experimental import pallas as pl
from jax.experimental.pallas import tpu as pltpu
from jax.experimental.pallas import tpu_sc as plsc

_B = 2
_NA = 16384
_NR = 2048
_CA = 128
_CS = 384

_BLK = 2048
_CHK = 256
_NAP = _NA + _BLK
_NT = 16
_APT = _NA // _NT
_NV = _APT // 16
_RPT = _NR // _NT
_QR = 32
_CG = _CS // 16


def _proj_body(x_ref, m_ref, g_ref, b_ref, w_ref, o_ref, carry_ref):
    k = pl.program_id(1)
    nblk = _NAP // _BLK

    @pl.when(k == 0)
    def _():
        carry_ref[...] = jnp.zeros_like(carry_ref)

    @pl.when(k < nblk - 1)
    def _():
        x = x_ref[0]
        mu = jnp.mean(x, axis=-1, keepdims=True)
        var = jnp.mean((x - mu) ** 2, axis=-1, keepdims=True)
        h = (x - mu) * lax.rsqrt(var + 1e-5) * g_ref[...] + b_ref[...]
        y = jnp.maximum(
            jnp.dot(h.astype(jnp.bfloat16), w_ref[...].astype(jnp.bfloat16),
                    preferred_element_type=jnp.float32),
            0.0)
        ii = lax.broadcasted_iota(jnp.int32, (_CHK, _CHK), 0)
        jj = lax.broadcasted_iota(jnp.int32, (_CHK, _CHK), 1)
        lt = (jj < ii).astype(jnp.float32)
        run = carry_ref[...]
        for ck in range(_BLK // _CHK):
            mrow = m_ref[0, ck:ck + 1, :]
            ltm = (lt * mrow).astype(jnp.bfloat16)
            ych = y[ck * _CHK:(ck + 1) * _CHK].astype(jnp.bfloat16)
            ex = jnp.dot(ltm, ych, preferred_element_type=jnp.float32)
            o_ref[0, ck * _CHK:(ck + 1) * _CHK, :] = ex + run
            run = run + (ex[_CHK - 1:_CHK, :]
                         + ych[_CHK - 1:_CHK, :].astype(jnp.float32)
                         * m_ref[0, ck:ck + 1, _CHK - 1:_CHK])
        carry_ref[...] = run

    @pl.when(k == nblk - 1)
    def _():
        o_ref[0] = jnp.broadcast_to(carry_ref[...], (_BLK, _CS))


def _prefix(x3d, mask3d, gamma, beta, w):
    nreal = _NA // _BLK - 1
    return pl.pallas_call(
        _proj_body,
        grid=(_B, _NAP // _BLK),
        in_specs=[
            pl.BlockSpec((1, _BLK, _CA),
                         lambda b, k: (b, jnp.minimum(k, nreal), 0)),
            pl.BlockSpec((1, _BLK // _CHK, _CHK),
                         lambda b, k: (b, jnp.minimum(k, nreal), 0)),
            pl.BlockSpec((1, _CA), lambda b, k: (0, 0)),
            pl.BlockSpec((1, _CA), lambda b, k: (0, 0)),
            pl.BlockSpec((_CA, _CS), lambda b, k: (0, 0)),
        ],
        out_specs=pl.BlockSpec((1, _BLK, _CS), lambda b, k: (b, k, 0)),
        out_shape=jax.ShapeDtypeStruct((_B, _NAP, _CS), jnp.float32),
        scratch_shapes=[pltpu.VMEM((1, _CS), jnp.float32)],
    )(x3d, mask3d, gamma, beta, w)


_mesh = plsc.VectorSubcoreMesh(
    core_axis_name="c", subcore_axis_name="s", num_cores=2, num_subcores=_NT
)

def _splat(vec, lane):
    return vec.at[jnp.full((16,), lane, jnp.int32)].get(
        mode="promise_in_bounds")


@functools.partial(
    pl.kernel,
    out_type=(
        jax.ShapeDtypeStruct((_B, _NR), jnp.int32),
        jax.ShapeDtypeStruct((_B, _NR), jnp.int32),
        jax.ShapeDtypeStruct((_B, _NR), jnp.float32),
    ),
    mesh=_mesh,
    compiler_params=pltpu.CompilerParams(needs_layout_passes=False),
    scratch_types=[
        pltpu.VMEM((_APT + 16,), jnp.int32),
        pltpu.VMEM((16,), jnp.int32),
        pltpu.VMEM((_NR,), jnp.int32),
        pltpu.VMEM((_NR,), jnp.int32),
        pltpu.VMEM((_NT, _RPT), jnp.int32),
        pltpu.VMEM((_NT, _RPT), jnp.int32),
        pltpu.VMEM((_RPT,), jnp.int32),
        pltpu.VMEM((_RPT,), jnp.int32),
        pltpu.VMEM((_RPT,), jnp.float32),
        pltpu.VMEM_SHARED((_NT, _NR), jnp.int32),
        pltpu.VMEM_SHARED((_NT, _NR), jnp.int32),
    ],
)
def _sc_tables(idx, gfo, glo, rco, idx_v, pn_v, f_loc, l_loc, fbuf, lbuf,
               gf, gl, recip, fsh, lsh):
    c = lax.axis_index("c")
    sid = lax.axis_index("s")
    i16 = jnp.arange(16, dtype=jnp.int32)
    base = sid * _APT
    r0 = sid * _RPT

    def _zero(i, _):
        z = jnp.zeros((16,), jnp.int32)
        f_loc[pl.ds(i * 16, 16)] = z
        l_loc[pl.ds(i * 16, 16)] = z
        return 0

    lax.fori_loop(0, _NR // 16, _zero, 0)

    pltpu.sync_copy(idx.at[c, pl.ds(base, _APT)], idx_v.at[pl.ds(0, _APT)])
    pn_v[...] = jnp.full((16,), -1, jnp.int32)
    idx_v[pl.ds(_APT, 16)] = jnp.full((16,), -2, jnp.int32)

    @pl.when(sid > 0)
    def _():
        pltpu.sync_copy(idx.at[c, pl.ds(base - 16, 16)], pn_v)

    @pl.when(sid < _NT - 1)
    def _():
        pltpu.sync_copy(idx.at[c, pl.ds(base + _APT, 16)],
                        idx_v.at[pl.ds(_APT, 16)])

    def _rdet(v, prev_vec):
        iv = idx_v[pl.ds(v * 16, 16)]
        nxt_vec = idx_v[pl.ds(v * 16 + 16, 16)]
        sh = iv.at[jnp.maximum(i16 - 1, 0)].get(mode="promise_in_bounds")
        sh = jnp.where(i16 == 0, _splat(prev_vec, 15), sh)
        nh = iv.at[jnp.minimum(i16 + 1, 15)].get(mode="promise_in_bounds")
        nh = jnp.where(i16 == 15, _splat(nxt_vec, 0), nh)
        posv = jnp.full((16,), base, jnp.int32) + v * 16 + i16
        plsc.store_scatter(f_loc, [iv], posv, mask=iv != sh)
        plsc.store_scatter(l_loc, [iv], posv + 1, mask=iv != nh)
        return iv

    lax.fori_loop(0, _NV, _rdet, pn_v[...])

    pltpu.sync_copy(f_loc, fsh.at[sid])
    pltpu.sync_copy(l_loc, lsh.at[sid])
    plsc.subcore_barrier()
    pltpu.sync_copy(fsh.at[:, pl.ds(r0, _RPT)], fbuf)
    pltpu.sync_copy(lsh.at[:, pl.ds(r0, _RPT)], lbuf)

    prow = c * _NAP

    def _combine(g, _):
        accf = jnp.zeros((16,), jnp.int32)
        accl = jnp.zeros((16,), jnp.int32)
        for t in range(_NT):
            accf = accf + fbuf[t, pl.ds(g * 16, 16)]
            accl = accl + lbuf[t, pl.ds(g * 16, 16)]
        cntf = (accl - accf).astype(jnp.float32)
        recip[pl.ds(g * 16, 16)] = 1.0 / (1.0 + cntf)
        off = jnp.full((16,), 0, jnp.int32) + prow
        gf[pl.ds(g * 16, 16)] = accf + off
        gl[pl.ds(g * 16, 16)] = accl + off
        return 0

    lax.fori_loop(0, _RPT // 16, _combine, 0)
    pltpu.sync_copy(gf, gfo.at[c, pl.ds(r0, _RPT)])
    pltpu.sync_copy(gl, glo.at[c, pl.ds(r0, _RPT)])
    pltpu.sync_copy(recip, rco.at[c, pl.ds(r0, _RPT)])


@functools.partial(
    pl.kernel,
    out_type=jax.ShapeDtypeStruct((_B, _NR, _CS), jnp.float32),
    mesh=_mesh,
    compiler_params=pltpu.CompilerParams(needs_layout_passes=False),
    scratch_types=[
        pltpu.VMEM((_RPT,), jnp.int32),
        pltpu.VMEM((_RPT,), jnp.int32),
        pltpu.VMEM((_RPT,), jnp.float32),
        pltpu.VMEM((_QR, _CS), jnp.float32),
        pltpu.VMEM((_QR, _CS), jnp.float32),
        pltpu.VMEM((_QR, _CS), jnp.float32),
        pltpu.VMEM((_QR, _CS), jnp.float32),
        pltpu.VMEM((_QR, _CS), jnp.float32),
        pltpu.VMEM((_QR, _CS), jnp.float32),
        pltpu.SemaphoreType.DMA,
        pltpu.SemaphoreType.DMA,
        pltpu.SemaphoreType.DMA,
        pltpu.SemaphoreType.DMA,
    ],
)
def _sc_finalize(p2d, s, gfo, glo, rco, out, gf, gl, recip,
                 pg1a, pg1b, pg2a, pg2b, sra, srb, semg0, semg1, semo0, semo1):
    c = lax.axis_index("c")
    sid = lax.axis_index("s")
    r0 = sid * _RPT
    pltpu.sync_copy(gfo.at[c, pl.ds(r0, _RPT)], gf)
    pltpu.sync_copy(glo.at[c, pl.ds(r0, _RPT)], gl)
    pltpu.sync_copy(rco.at[c, pl.ds(r0, _RPT)], recip)

    bufs = ((pg1a, pg2a, sra, semg0, semo0), (pg1b, pg2b, srb, semg1, semo1))
    nq = _RPT // _QR

    def _issue(q):
        b1, b2, bs, sg, _ = bufs[q % 2]
        return (
            pltpu.async_copy(p2d.at[gf.at[pl.ds(q * _QR, _QR)]], b1, sg),
            pltpu.async_copy(p2d.at[gl.at[pl.ds(q * _QR, _QR)]], b2, sg),
            pltpu.async_copy(s.at[c, pl.ds(r0 + q * _QR, _QR)], bs, sg),
        )

    pend = {0: _issue(0)}
    outh = {}
    for q in range(nq):
        b1, b2, bs, _, so = bufs[q % 2]
        if q + 1 < nq:
            if (q + 1) % 2 in outh:
                outh.pop((q + 1) % 2).wait()
            pend[q + 1] = _issue(q + 1)
        for hh in pend.pop(q):
            hh.wait()

        @plsc.parallel_loop(0, _QR, step=2, unroll=2)
        def _rowfin(i, q=q, b1=b1, b2=b2, bs=bs):
            for u in range(2):
                r = i + u
                qq = q * _QR + r
                rv = recip[pl.ds((qq // 16) * 16, 16)]
                rs = rv.at[jnp.full((16,), qq % 16, jnp.int32)].get(
                    mode="promise_in_bounds")
                for cg in range(_CG):
                    sl = pl.ds(cg * 16, 16)
                    bs[r, sl] = (bs[r, sl] + b2[r, sl] - b1[r, sl]) * rs
        outh[q % 2] = pltpu.async_copy(bs, out.at[c, pl.ds(r0 + q * _QR, _QR)],
                                       so)
    for hh in outh.values():
        hh.wait()


def kernel(atom_features, s, atom_res_idx, atom_mask, ln_gamma, ln_beta, W):
    idx = atom_res_idx.astype(jnp.int32)
    gfo, glo, rco = _sc_tables(idx)
    p = _prefix(atom_features, atom_mask.reshape(_B, _NA // _CHK, _CHK),
                ln_gamma.reshape(1, _CA), ln_beta.reshape(1, _CA), W)
    return _sc_finalize(p.reshape(_B * _NAP, _CS), s, gfo, glo, rco)
